# baseline (speedup 1.0000x reference)
import functools

import jax
import jax.numpy as jnp
from jax import lax
from jax.experimental import pallas as pl
from jax.experimental.pallas import tpu as pltpu
from jax.experimental.pallas import tpu_sc as plsc

_N = 8192
_QBLK = 1024
_CH = 4096
_LANES = 128
_NW = 32
_BPW = _N // _NW
_L = 16


def _select_body(p_ref, gt_t_ref, r_ref, t_ref, out_ref):
    p = p_ref[...]
    r = r_ref[...]
    q = jnp.dot(p, r.T, preferred_element_type=jnp.float32) + t_ref[...]
    qb = q.astype(jnp.bfloat16)
    lhs = jnp.concatenate(
        [-2.0 * qb, jnp.ones((_QBLK, 3), jnp.bfloat16)], axis=1)

    gt = gt_t_ref[...]
    gb = gt.astype(jnp.bfloat16)
    gg = gt[0:1, :] * gt[0:1, :] + gt[1:2, :] * gt[1:2, :] + gt[2:3, :] * gt[2:3, :]
    gh = gg.astype(jnp.bfloat16)
    r1 = gg - gh.astype(jnp.float32)
    gm = r1.astype(jnp.bfloat16)
    gl = (r1 - gm.astype(jnp.float32)).astype(jnp.bfloat16)
    rhs = jnp.concatenate([gb, gh, gm, gl], axis=0)

    dn = (((1,), (0,)), ((), ()))
    inf = jnp.float32(jnp.inf)
    smin = jnp.full((_QBLK, _LANES), inf, dtype=jnp.float32)
    sid = jnp.zeros((_QBLK, _LANES), dtype=jnp.float32)
    for c in range(_N // _CH):
        lo, hi = c * _CH, (c + 1) * _CH
        s = lax.dot_general(lhs, rhs[:, lo:hi], dn,
                            preferred_element_type=jnp.float32)
        for k in range(_CH // _LANES):
            ss = s[:, k * _LANES:(k + 1) * _LANES]
            upd = ss < smin
            smin = jnp.where(upd, ss, smin)
            sid = jnp.where(upd, jnp.float32(c * (_CH // _LANES) + k), sid)
    srow = jnp.min(smin, axis=1, keepdims=True)
    lane = lax.broadcasted_iota(jnp.int32, (_QBLK, _LANES), 1).astype(jnp.float32)
    code = sid * jnp.float32(_LANES) + lane
    j = jnp.min(jnp.where(smin == srow, code, jnp.float32(2 * _N)),
                axis=1, keepdims=True)
    out_ref[...] = jnp.concatenate([j, q], axis=1)


def _sqrt16(x):
    i = lax.bitcast_convert_type(x, jnp.int32)
    y = lax.bitcast_convert_type(jnp.int32(0x5F3759DF) - (i >> 1), jnp.float32)
    for _ in range(4):
        y = y * (1.5 - 0.5 * x * y * y)
    return jnp.where(x < 1e-35, 0.0, x * y)


def _distance_body(gt_hbm, rec_hbm, syms_hbm, asyms_hbm,
                   list_v, rec_v, jlist_v, gsel_v, glin_v, acc_v,
                   sem_d, sem_g, sem_l):
    wid = lax.axis_index("s") * 2 + lax.axis_index("c")
    base = wid * _BPW
    b4 = base * 4
    ramp = lax.iota(jnp.int32, _L)
    ramp4 = ramp * 4

    for f in range(4):
        for c in range(_BPW // _L):
            list_v[pl.ds(f * _BPW + c * _L, _L)] = (
                ramp4 + jnp.int32(b4 + f + 4 * _L * c))
    deint = pltpu.async_copy(rec_hbm.at[list_v], rec_v, sem_d)
    l1 = pltpu.async_copy(gt_hbm.at[pl.ds(base, _BPW)],
                          glin_v.at[pl.ds(0, _BPW)], sem_l)
    l2 = pltpu.async_copy(gt_hbm.at[pl.ds(_N + base, _BPW)],
                          glin_v.at[pl.ds(_BPW, _BPW)], sem_l)
    l3 = pltpu.async_copy(gt_hbm.at[pl.ds(2 * _N + base, _BPW)],
                          glin_v.at[pl.ds(2 * _BPW, _BPW)], sem_l)
    deint.wait()

    for c in range(_BPW // _L):
        jv = rec_v[pl.ds(c * _L, _L)].astype(jnp.int32)
        jlist_v[pl.ds(c * _L, _L)] = jv
        jlist_v[pl.ds(_BPW + c * _L, _L)] = jv + jnp.int32(_N)
        jlist_v[pl.ds(2 * _BPW + c * _L, _L)] = jv + jnp.int32(2 * _N)
    gs = pltpu.async_copy(gt_hbm.at[jlist_v], gsel_v, sem_g)
    gs.wait()
    l1.wait(); l2.wait(); l3.wait()

    acc_s = jnp.zeros((_L,), jnp.float32)
    acc_a = jnp.zeros((_L,), jnp.float32)
    for c in range(_BPW // _L):
        o = c * _L
        qx = rec_v[pl.ds(_BPW + o, _L)]
        qy = rec_v[pl.ds(2 * _BPW + o, _L)]
        qz = rec_v[pl.ds(3 * _BPW + o, _L)]
        dx = qx - gsel_v[pl.ds(o, _L)]
        dy = qy - gsel_v[pl.ds(_BPW + o, _L)]
        dz = qz - gsel_v[pl.ds(2 * _BPW + o, _L)]
        acc_s = acc_s + _sqrt16(dx * dx + dy * dy + dz * dz)
        ax = qx - glin_v[pl.ds(o, _L)]
        ay = qy - glin_v[pl.ds(_BPW + o, _L)]
        az = qz - glin_v[pl.ds(2 * _BPW + o, _L)]
        acc_a = acc_a + _sqrt16(ax * ax + ay * ay + az * az)

    acc_v[...] = acc_s
    pltpu.sync_copy(acc_v, syms_hbm.at[wid])
    acc_v[...] = acc_a
    pltpu.sync_copy(acc_v, asyms_hbm.at[wid])


def kernel(pred_R, pred_t, pts_model, pts_gt, model_index, device):
    P = pts_model[0]
    G = pts_gt[0]
    GT = G.T
    R = pred_R[0]
    T = pred_t

    rec = pl.pallas_call(
        _select_body,
        grid=(_N // _QBLK,),
        in_specs=[
            pl.BlockSpec((_QBLK, 3), lambda i: (i, 0)),
            pl.BlockSpec((3, _N), lambda i: (0, 0)),
            pl.BlockSpec((3, 3), lambda i: (0, 0)),
            pl.BlockSpec((1, 3), lambda i: (0, 0)),
        ],
        out_specs=pl.BlockSpec((_QBLK, 4), lambda i: (i, 0)),
        out_shape=jax.ShapeDtypeStruct((_N, 4), jnp.float32),
    )(P, GT, R, T)

    distance = functools.partial(
        pl.kernel,
        mesh=plsc.VectorSubcoreMesh(core_axis_name="c", subcore_axis_name="s"),
        out_type=[
            jax.ShapeDtypeStruct((_NW, _L), jnp.float32),
            jax.ShapeDtypeStruct((_NW, _L), jnp.float32),
        ],
        scratch_types=[
            pltpu.VMEM((_BPW * 4,), jnp.int32),
            pltpu.VMEM((_BPW * 4,), jnp.float32),
            pltpu.VMEM((_BPW * 3,), jnp.int32),
            pltpu.VMEM((_BPW * 3,), jnp.float32),
            pltpu.VMEM((_BPW * 3,), jnp.float32),
            pltpu.VMEM((_L,), jnp.float32),
            pltpu.SemaphoreType.DMA,
            pltpu.SemaphoreType.DMA,
            pltpu.SemaphoreType.DMA,
        ],
    )(_distance_body)
    sym_parts, asym_parts = distance(GT.reshape(3 * _N), rec.reshape(4 * _N))

    is_sym = model_index.reshape(-1)[0] == 0
    loss = jnp.where(is_sym, jnp.sum(sym_parts), jnp.sum(asym_parts)) / _N
    return loss.reshape(1)


# speedup vs baseline: 1.0027x; 1.0027x over previous
import functools

import jax
import jax.numpy as jnp
from jax import lax
from jax.experimental import pallas as pl
from jax.experimental.pallas import tpu as pltpu
from jax.experimental.pallas import tpu_sc as plsc

_N = 8192
_QBLK = 1024
_CH = 2048
_LANES = 128
_NW = 32
_BPW = _N // _NW
_L = 16


def _select_body(p_ref, gt_t_ref, r_ref, t_ref, out_ref):
    p = p_ref[...]
    r = r_ref[...]
    q = jnp.dot(p, r.T, preferred_element_type=jnp.float32) + t_ref[...]
    qb = q.astype(jnp.bfloat16)
    lhs = jnp.concatenate(
        [-2.0 * qb, jnp.ones((_QBLK, 3), jnp.bfloat16)], axis=1)

    gt = gt_t_ref[...]
    gb = gt.astype(jnp.bfloat16)
    gg = gt[0:1, :] * gt[0:1, :] + gt[1:2, :] * gt[1:2, :] + gt[2:3, :] * gt[2:3, :]
    gh = gg.astype(jnp.bfloat16)
    r1 = gg - gh.astype(jnp.float32)
    gm = r1.astype(jnp.bfloat16)
    gl = (r1 - gm.astype(jnp.float32)).astype(jnp.bfloat16)
    rhs = jnp.concatenate([gb, gh, gm, gl], axis=0)

    dn = (((1,), (0,)), ((), ()))
    inf = jnp.float32(jnp.inf)
    smin = jnp.full((_QBLK, _LANES), inf, dtype=jnp.float32)
    sid = jnp.zeros((_QBLK, _LANES), dtype=jnp.float32)
    for c in range(_N // _CH):
        lo, hi = c * _CH, (c + 1) * _CH
        s = lax.dot_general(lhs, rhs[:, lo:hi], dn,
                            preferred_element_type=jnp.float32)
        for k in range(_CH // _LANES):
            ss = s[:, k * _LANES:(k + 1) * _LANES]
            upd = ss < smin
            smin = jnp.where(upd, ss, smin)
            sid = jnp.where(upd, jnp.float32(c * (_CH // _LANES) + k), sid)
    srow = jnp.min(smin, axis=1, keepdims=True)
    lane = lax.broadcasted_iota(jnp.int32, (_QBLK, _LANES), 1).astype(jnp.float32)
    code = sid * jnp.float32(_LANES) + lane
    j = jnp.min(jnp.where(smin == srow, code, jnp.float32(2 * _N)),
                axis=1, keepdims=True)
    out_ref[...] = jnp.concatenate([j, q], axis=1)


def _sqrt16(x):
    i = lax.bitcast_convert_type(x, jnp.int32)
    y = lax.bitcast_convert_type(jnp.int32(0x5F3759DF) - (i >> 1), jnp.float32)
    for _ in range(4):
        y = y * (1.5 - 0.5 * x * y * y)
    return jnp.where(x < 1e-35, 0.0, x * y)


def _distance_body(gt_hbm, rec_hbm, syms_hbm, asyms_hbm,
                   list_v, rec_v, jlist_v, gsel_v, glin_v, acc_v,
                   sem_d, sem_g, sem_l):
    wid = lax.axis_index("s") * 2 + lax.axis_index("c")
    base = wid * _BPW
    b4 = base * 4
    ramp = lax.iota(jnp.int32, _L)
    ramp4 = ramp * 4

    for f in range(4):
        for c in range(_BPW // _L):
            list_v[pl.ds(f * _BPW + c * _L, _L)] = (
                ramp4 + jnp.int32(b4 + f + 4 * _L * c))
    deint = pltpu.async_copy(rec_hbm.at[list_v], rec_v, sem_d)
    l1 = pltpu.async_copy(gt_hbm.at[pl.ds(base, _BPW)],
                          glin_v.at[pl.ds(0, _BPW)], sem_l)
    l2 = pltpu.async_copy(gt_hbm.at[pl.ds(_N + base, _BPW)],
                          glin_v.at[pl.ds(_BPW, _BPW)], sem_l)
    l3 = pltpu.async_copy(gt_hbm.at[pl.ds(2 * _N + base, _BPW)],
                          glin_v.at[pl.ds(2 * _BPW, _BPW)], sem_l)
    deint.wait()

    for c in range(_BPW // _L):
        jv = rec_v[pl.ds(c * _L, _L)].astype(jnp.int32)
        jlist_v[pl.ds(c * _L, _L)] = jv
        jlist_v[pl.ds(_BPW + c * _L, _L)] = jv + jnp.int32(_N)
        jlist_v[pl.ds(2 * _BPW + c * _L, _L)] = jv + jnp.int32(2 * _N)
    gs = pltpu.async_copy(gt_hbm.at[jlist_v], gsel_v, sem_g)
    gs.wait()
    l1.wait(); l2.wait(); l3.wait()

    acc_s = jnp.zeros((_L,), jnp.float32)
    acc_a = jnp.zeros((_L,), jnp.float32)
    for c in range(_BPW // _L):
        o = c * _L
        qx = rec_v[pl.ds(_BPW + o, _L)]
        qy = rec_v[pl.ds(2 * _BPW + o, _L)]
        qz = rec_v[pl.ds(3 * _BPW + o, _L)]
        dx = qx - gsel_v[pl.ds(o, _L)]
        dy = qy - gsel_v[pl.ds(_BPW + o, _L)]
        dz = qz - gsel_v[pl.ds(2 * _BPW + o, _L)]
        acc_s = acc_s + _sqrt16(dx * dx + dy * dy + dz * dz)
        ax = qx - glin_v[pl.ds(o, _L)]
        ay = qy - glin_v[pl.ds(_BPW + o, _L)]
        az = qz - glin_v[pl.ds(2 * _BPW + o, _L)]
        acc_a = acc_a + _sqrt16(ax * ax + ay * ay + az * az)

    acc_v[...] = acc_s
    pltpu.sync_copy(acc_v, syms_hbm.at[wid])
    acc_v[...] = acc_a
    pltpu.sync_copy(acc_v, asyms_hbm.at[wid])


def kernel(pred_R, pred_t, pts_model, pts_gt, model_index, device):
    P = pts_model[0]
    G = pts_gt[0]
    GT = G.T
    R = pred_R[0]
    T = pred_t

    rec = pl.pallas_call(
        _select_body,
        grid=(_N // _QBLK,),
        in_specs=[
            pl.BlockSpec((_QBLK, 3), lambda i: (i, 0)),
            pl.BlockSpec((3, _N), lambda i: (0, 0)),
            pl.BlockSpec((3, 3), lambda i: (0, 0)),
            pl.BlockSpec((1, 3), lambda i: (0, 0)),
        ],
        out_specs=pl.BlockSpec((_QBLK, 4), lambda i: (i, 0)),
        out_shape=jax.ShapeDtypeStruct((_N, 4), jnp.float32),
        compiler_params=pltpu.CompilerParams(
            dimension_semantics=("arbitrary",)),
    )(P, GT, R, T)

    distance = functools.partial(
        pl.kernel,
        mesh=plsc.VectorSubcoreMesh(core_axis_name="c", subcore_axis_name="s"),
        out_type=[
            jax.ShapeDtypeStruct((_NW, _L), jnp.float32),
            jax.ShapeDtypeStruct((_NW, _L), jnp.float32),
        ],
        scratch_types=[
            pltpu.VMEM((_BPW * 4,), jnp.int32),
            pltpu.VMEM((_BPW * 4,), jnp.float32),
            pltpu.VMEM((_BPW * 3,), jnp.int32),
            pltpu.VMEM((_BPW * 3,), jnp.float32),
            pltpu.VMEM((_BPW * 3,), jnp.float32),
            pltpu.VMEM((_L,), jnp.float32),
            pltpu.SemaphoreType.DMA,
            pltpu.SemaphoreType.DMA,
            pltpu.SemaphoreType.DMA,
        ],
    )(_distance_body)
    sym_parts, asym_parts = distance(GT.reshape(3 * _N), rec.reshape(4 * _N))

    is_sym = model_index.reshape(-1)[0] == 0
    loss = jnp.where(is_sym, jnp.sum(sym_parts), jnp.sum(asym_parts)) / _N
    return loss.reshape(1)
